# baseline (speedup 1.0000x reference)
import functools

import jax
import jax.numpy as jnp
from jax import lax
from jax.experimental import pallas as pl
from jax.experimental.pallas import tpu as pltpu
from jax.experimental.pallas import tpu_sc as plsc

N = 10000
E = 320000
G = 100
K = 30
NPAD = 10240
GPAD = 128
NC = 2
NS = 16
NW = NC * NS
EROWS = 2560
EPAD = EROWS * 128
ROWS_PER_W = EROWS // NW
NROWS_PER_TILE = NPAD // NS


@functools.lru_cache(maxsize=None)
def _mesh():
    return plsc.VectorSubcoreMesh(core_axis_name="c", subcore_axis_name="s",
                                  num_cores=NC, num_subcores=NS)


def _zero_vmem_2d(ref, nrows, width):
    z16 = jnp.zeros((16,), jnp.float32)

    def body(i, _):
        for c0 in range(0, width, 16):
            ref[i, pl.ds(c0, 16)] = z16
        return 0

    lax.fori_loop(0, nrows, body, 0)


def _zero_vmem_1d_i32(ref, nelems):
    z16 = jnp.zeros((16,), jnp.int32)

    def body(i, _):
        ref[pl.ds(i * 16, 16)] = z16
        return 0

    lax.fori_loop(0, nelems // 16, body, 0)


_EREAL = E // 128


@functools.lru_cache(maxsize=None)
def _get_sc_pre():
    @functools.partial(
        pl.kernel,
        out_type=(
            jax.ShapeDtypeStruct((EROWS, 128), jnp.int32),
            jax.ShapeDtypeStruct((EROWS, 128), jnp.int32),
            jax.ShapeDtypeStruct((NW, NPAD), jnp.int32),
            jax.ShapeDtypeStruct((GPAD,), jnp.int32),
        ),
        mesh=_mesh(),
        compiler_params=pltpu.CompilerParams(needs_layout_passes=False, use_tc_tiling_on_sc=False),
        scratch_types=[
            pltpu.VMEM((ROWS_PER_W, 128), jnp.int32),
            pltpu.VMEM((ROWS_PER_W, 128), jnp.int32),
            pltpu.VMEM((NPAD,), jnp.int32),
            pltpu.VMEM((NPAD,), jnp.int32),
            pltpu.VMEM((GPAD,), jnp.int32),
        ],
    )
    def sc_pre(ei_hbm, batch_hbm, rowp_hbm, colp_hbm, degs_hbm, cnt_hbm,
               rowb, colb, degb, batchb, cntb):
        c = lax.axis_index("c")
        s = lax.axis_index("s")
        wid = s * NC + c
        base = wid * ROWS_PER_W
        nreal = _EREAL - 31 * ROWS_PER_W

        _zero_vmem_1d_i32(degb, NPAD)

        iota = lax.iota(jnp.int32, 16)
        dummy = N + iota
        ones = jnp.ones((16,), jnp.int32)

        @pl.when(wid < NW - 1)
        def _():
            pltpu.sync_copy(ei_hbm.at[0, pl.ds(base, ROWS_PER_W)], rowb)
            pltpu.sync_copy(ei_hbm.at[1, pl.ds(base, ROWS_PER_W)], colb)

        @pl.when(wid == NW - 1)
        def _():
            pltpu.sync_copy(ei_hbm.at[0, pl.ds(base, nreal)],
                            rowb.at[pl.ds(0, nreal)])
            pltpu.sync_copy(ei_hbm.at[1, pl.ds(base, nreal)],
                            colb.at[pl.ds(0, nreal)])

            def padrow(j, _):
                for k in range(8):
                    v = ((base + j) * 128 + k * 16 + iota) % N
                    rowb[j, pl.ds(k * 16, 16)] = v
                    colb[j, pl.ds(k * 16, 16)] = v
                return 0

            lax.fori_loop(nreal, ROWS_PER_W, padrow, 0)

        def erow(j, _):
            for k in range(8):
                rv = rowb[j, pl.ds(k * 16, 16)]
                cv = colb[j, pl.ds(k * 16, 16)]
                cpv = jnp.where(rv == cv, dummy, cv)
                colb[j, pl.ds(k * 16, 16)] = cpv
                plsc.addupdate_scatter(degb, [cpv], ones)
            return 0

        lax.fori_loop(0, ROWS_PER_W, erow, 0)

        pltpu.sync_copy(rowb, rowp_hbm.at[pl.ds(base, ROWS_PER_W)])
        pltpu.sync_copy(colb, colp_hbm.at[pl.ds(base, ROWS_PER_W)])
        pltpu.sync_copy(degb, degs_hbm.at[wid])

        @pl.when(jnp.logical_and(c == 0, s == 0))
        def _():
            _zero_vmem_1d_i32(cntb, GPAD)
            pltpu.sync_copy(batch_hbm, batchb.at[pl.ds(0, N)])

            def brow(i, _):
                bv = batchb[pl.ds(i * 16, 16)]
                plsc.addupdate_scatter(cntb, [bv], ones)
                return 0

            lax.fori_loop(0, N // 16, brow, 0)
            pltpu.sync_copy(cntb, cnt_hbm)

    return sc_pre


_GK = 10
_NGO = ROWS_PER_W // (2 * _GK)
_ZR = 80


@functools.lru_cache(maxsize=None)
def _get_propagate(F):
    @functools.partial(
        pl.kernel,
        out_type=jax.ShapeDtypeStruct((NC, NPAD, F), jnp.float32),
        mesh=_mesh(),
        compiler_params=pltpu.CompilerParams(needs_layout_passes=False, use_tc_tiling_on_sc=False),
        scratch_types=[
            pltpu.VMEM((ROWS_PER_W, 128), jnp.int32),
            pltpu.VMEM((ROWS_PER_W, 128), jnp.int32),
            pltpu.VMEM((2, _GK, 128, F), jnp.float32),
            pltpu.VMEM((_ZR, F), jnp.float32),
            pltpu.VMEM_SHARED((NPAD, F), jnp.float32),
            pltpu.SemaphoreType.DMA,
            pltpu.SemaphoreType.DMA,
        ],
    )
    def prop(z_hbm, row_hbm, colp_hbm, out_hbm, rowb, colb, gbuf, zbuf, acc,
             semg, sems):
        c = lax.axis_index("c")
        s = lax.axis_index("s")
        wid = s * NC + c
        base = wid * ROWS_PER_W
        stripe = pl.ds(s * NROWS_PER_TILE, NROWS_PER_TILE)

        _zero_vmem_2d(zbuf, _ZR, F)
        for q in range(NROWS_PER_TILE // _ZR):
            pltpu.sync_copy(
                zbuf, acc.at[pl.ds(s * NROWS_PER_TILE + q * _ZR, _ZR)])

        pltpu.sync_copy(row_hbm.at[pl.ds(base, ROWS_PER_W)], rowb)
        pltpu.sync_copy(colp_hbm.at[pl.ds(base, ROWS_PER_W)], colb)
        plsc.subcore_barrier()

        def start_gathers(grp, half):
            for k in range(_GK):
                pltpu.async_copy(z_hbm.at[rowb.at[grp * _GK + k]],
                                 gbuf.at[half, k], semg)

        def wait_gathers(grp, half):
            for k in range(_GK):
                pltpu.make_async_copy(z_hbm.at[rowb.at[grp * _GK + k]],
                                      gbuf.at[half, k], semg).wait()

        def fire_scatters(grp, half):
            for k in range(_GK):
                pltpu.async_copy(gbuf.at[half, k],
                                 acc.at[colb.at[grp * _GK + k]], sems,
                                 add=True)

        def drain_scatters(grp, half):
            for k in range(_GK):
                pltpu.make_async_copy(gbuf.at[half, k],
                                      acc.at[colb.at[grp * _GK + k]],
                                      sems).wait()

        start_gathers(0, 0)

        def outer(go, _):
            ga = 2 * go
            gb = 2 * go + 1
            wait_gathers(ga, 0)

            @pl.when(go > 0)
            def _():
                drain_scatters(ga - 1, 1)

            start_gathers(gb, 1)
            fire_scatters(ga, 0)
            wait_gathers(gb, 1)
            drain_scatters(ga, 0)

            @pl.when(go < _NGO - 1)
            def _():
                start_gathers(ga + 2, 0)

            fire_scatters(gb, 1)
            return 0

        lax.fori_loop(0, _NGO, outer, 0)
        drain_scatters(2 * _NGO - 1, 1)

        plsc.subcore_barrier()
        pltpu.sync_copy(acc.at[stripe], out_hbm.at[c, stripe])

    return prop


_NG_PER_W = (G + NW - 1) // NW


@functools.lru_cache(maxsize=None)
def _get_sc_topk():
    @functools.partial(
        pl.kernel,
        out_type=(
            jax.ShapeDtypeStruct((G, 16, 128), jnp.float32),
            jax.ShapeDtypeStruct((G, 16, 128), jnp.float32),
        ),
        mesh=_mesh(),
        compiler_params=pltpu.CompilerParams(needs_layout_passes=False, use_tc_tiling_on_sc=False),
        scratch_types=[
            pltpu.VMEM((NPAD,), jnp.float32),
            pltpu.VMEM((GPAD,), jnp.int32),
            pltpu.VMEM((GPAD,), jnp.int32),
            pltpu.VMEM((2, 16), jnp.int32),
            pltpu.VMEM((2, 16, 128), jnp.float32),
            pltpu.SemaphoreType.DMA,
        ],
    )
    def sc_topk(key_hbm, cnt_hbm, xc_hbm, pe_hbm, po_hbm,
                keyb, startb, cntb, idxb, gbuf, sem):
        c = lax.axis_index("c")
        s = lax.axis_index("s")
        wid = s * NC + c

        pltpu.sync_copy(key_hbm, keyb)
        pltpu.sync_copy(cnt_hbm, cntb)

        iota = lax.iota(jnp.int32, 16)

        def scan_chunk(i, carry):
            cv = cntb[pl.ds(i * 16, 16)]
            cs = plsc.cumsum(cv)
            startb[pl.ds(i * 16, 16)] = carry + cs - cv
            return carry + jnp.sum(jnp.where(iota == 15, cs, 0))

        lax.fori_loop(0, GPAD // 16, scan_chunk, jnp.int32(0))

        def do_graph(gl, _):
            g = wid + NW * gl

            @pl.when(g < G)
            def _():
                sel = iota == (g % 16)
                start = jnp.sum(jnp.where(sel, startb[pl.ds((g // 16) * 16, 16)], 0))
                cnt = jnp.sum(jnp.where(sel, cntb[pl.ds((g // 16) * 16, 16)], 0))
                nch = (cnt + 15) // 16

                idxb[0, pl.ds(0, 16)] = N + iota
                idxb[1, pl.ds(0, 16)] = N + iota
                def ichunk(i, _):
                    a = keyb[pl.ds(start + i * 16, 16)]
                    ii = i * 16 + iota
                    validi = ii < cnt
                    rank = jnp.zeros((16,), jnp.int32)

                    def jlt(jc, r):
                        b = keyb[pl.ds(start + jc * 16, 16)]
                        for l in range(16):
                            r = r + jnp.where(b[l] >= a, 1, 0)
                        return r

                    rank = lax.fori_loop(0, i, jlt, rank)

                    for l in range(16):
                        bl = a[l]
                        jj = i * 16 + l
                        take = ((bl > a) | ((bl == a) & (iota > l))) & (jj < cnt)
                        rank = rank + jnp.where(take, 1, 0)

                    def jgt(jc, r):
                        b = keyb[pl.ds(start + jc * 16, 16)]
                        for l in range(16):
                            r = r + jnp.where(b[l] > a, 1, 0)
                        return r

                    rank = lax.fori_loop(i + 1, nch - 1, jgt, rank)

                    def last(r):
                        b = keyb[pl.ds(start + (nch - 1) * 16, 16)]
                        for l in range(16):
                            jj = (nch - 1) * 16 + l
                            r = r + jnp.where((b[l] > a) & (jj < cnt), 1, 0)
                        return r

                    rank = lax.cond(i < nch - 1, last, lambda r: r, rank)
                    node = start + ii
                    half = rank // 2
                    even = (rank % 2) == 0
                    mask_sel = validi & (rank < K)
                    plsc.store_scatter(idxb, [jnp.zeros((16,), jnp.int32), half],
                                       node, mask=mask_sel & even)
                    plsc.store_scatter(idxb, [jnp.ones((16,), jnp.int32), half],
                                       node, mask=mask_sel & (~even))
                    return 0

                lax.fori_loop(0, nch, ichunk, 0)

                he = pltpu.async_copy(xc_hbm.at[idxb.at[0]], gbuf.at[0], sem)
                ho = pltpu.async_copy(xc_hbm.at[idxb.at[1]], gbuf.at[1], sem)
                he.wait()
                pltpu.sync_copy(gbuf.at[0], pe_hbm.at[g])
                ho.wait()
                pltpu.sync_copy(gbuf.at[1], po_hbm.at[g])

            return 0

        lax.fori_loop(0, _NG_PER_W, do_graph, 0)

    return sc_topk


def _tc_xw_body(x_ref, w1_ref, xw_ref):
    xw_ref[...] = jnp.dot(x_ref[...], w1_ref[...],
                          preferred_element_type=jnp.float32)


_tc_xw = pl.pallas_call(
    _tc_xw_body,
    out_shape=jax.ShapeDtypeStruct((N, 32), jnp.float32),
)


def _tc1_body(xw_ref, degs_ref, z1_ref, dis_ref):
    deg = jnp.sum(degs_ref[:, :N].astype(jnp.float32), axis=0) + 1.0
    dis = lax.rsqrt(deg)[:, None]
    dis_ref[...] = dis
    z1_ref[:N, :] = xw_ref[...] * dis
    z1_ref[N:, :] = jnp.zeros((NPAD - N, 32), jnp.float32)


_tc1 = pl.pallas_call(
    _tc1_body,
    out_shape=(
        jax.ShapeDtypeStruct((NPAD, 32), jnp.float32),
        jax.ShapeDtypeStruct((N, 1), jnp.float32),
    ),
)


def _tc_mid_body(accs_ref, z_ref, b_ref, w_ref, dis_ref, h_ref, znext_ref, *, fout):
    acc = accs_ref[0, :N, :] + accs_ref[1, :N, :] + z_ref[:N, :]
    h = jnp.tanh(dis_ref[...] * acc + b_ref[...][None, :])
    h_ref[...] = h
    z = jnp.dot(h, w_ref[...], preferred_element_type=jnp.float32) * dis_ref[...]
    if fout == 16:
        znext_ref[:N, :] = jnp.broadcast_to(z[:, :1], (N, 16))
        znext_ref[N:, :] = jnp.zeros((NPAD - N, 16), jnp.float32)
    else:
        znext_ref[:N, :] = z
        znext_ref[N:, :] = jnp.zeros((NPAD - N, 32), jnp.float32)


def _make_tc_mid(fin, fout):
    return pl.pallas_call(
        functools.partial(_tc_mid_body, fout=fout),
        out_shape=(
            jax.ShapeDtypeStruct((N, fin), jnp.float32),
            jax.ShapeDtypeStruct((NPAD, fout), jnp.float32),
        ),
    )


_tc2 = _make_tc_mid(32, 32)
_tc4 = _make_tc_mid(32, 16)


def _tc5_body(accs_ref, z4_ref, b4_ref, dis_ref, h1_ref, h2_ref, h3_ref,
              xc_ref, key_ref):
    acc = accs_ref[0, :N, :1] + accs_ref[1, :N, :1] + z4_ref[:N, :1]
    h4 = jnp.tanh(dis_ref[...] * acc + b4_ref[...][None, :])
    xc_ref[:N, 0:32] = h1_ref[...]
    xc_ref[:N, 32:64] = h2_ref[...]
    xc_ref[:N, 64:96] = h3_ref[...]
    xc_ref[:N, 96:128] = jnp.concatenate(
        [h4, jnp.zeros((N, 31), jnp.float32)], axis=1)
    xc_ref[N:NPAD, :] = jnp.zeros((NPAD - N, 128), jnp.float32)
    key_ref[pl.ds(0, N)] = h4[:, 0]
    key_ref[pl.ds(N, NPAD - N)] = jnp.zeros((NPAD - N,), jnp.float32)


_tc5 = pl.pallas_call(
    _tc5_body,
    out_shape=(
        jax.ShapeDtypeStruct((NPAD, 128), jnp.float32),
        jax.ShapeDtypeStruct((NPAD,), jnp.float32),
    ),
)


def _tc_head_body(pe_ref, po_ref, c5_ref, c5b_ref, w6_ref, c6b_ref,
                  f1_ref, f1b_ref, f2_ref, f2b_ref, out_ref):
    ae = jnp.maximum(jnp.dot(pe_ref[...], c5_ref[...],
                             preferred_element_type=jnp.float32)
                     + c5b_ref[...][None, :], 0.0)
    ao = jnp.maximum(jnp.dot(po_ref[...], c5_ref[...],
                             preferred_element_type=jnp.float32)
                     + c5b_ref[...][None, :], 0.0)
    b3 = jnp.maximum(ae, ao).reshape(G, 16, 16)
    h1 = jnp.zeros((G, 128), jnp.float32)
    for t in range(11):
        a = jnp.zeros((G, 32), jnp.float32)
        for dt in range(5):
            a = a + jnp.dot(b3[:, t + dt, :],
                            w6_ref[pl.ds(dt * 16, 16), :],
                            preferred_element_type=jnp.float32)
        c2t = jnp.maximum(a + c6b_ref[...][None, :], 0.0)
        h1 = h1 + jnp.dot(c2t, f1_ref[pl.ds(t * 32, 32), :],
                          preferred_element_type=jnp.float32)
    h = jnp.maximum(h1 + f1b_ref[...][None, :], 0.0)
    logits = jnp.dot(h, f2_ref[...],
                     preferred_element_type=jnp.float32) + f2b_ref[...][None, :]
    m = jnp.max(logits, axis=-1, keepdims=True)
    lse = m + jnp.log(jnp.sum(jnp.exp(logits - m), axis=-1, keepdims=True))
    out_ref[...] = logits - lse


_tc_head = pl.pallas_call(
    _tc_head_body,
    out_shape=jax.ShapeDtypeStruct((G, 2), jnp.float32),
)


def kernel(x, edge_index, batch, W1, b1, W2, b2, W3, b3, W4, b4,
           c5w, c5b, c6w, c6b, f1w, f1b, f2w, f2b):
    ei3 = edge_index.reshape(2, E // 128, 128)

    xw1 = _tc_xw(x, W1)
    rowp, colp, degs, cnt = _get_sc_pre()(ei3, batch)
    z1, dis = _tc1(xw1, degs)

    prop32 = _get_propagate(32)
    acc1 = prop32(z1, rowp, colp)
    h1, z2 = _tc2(acc1, z1, b1, W2, dis)
    acc2 = prop32(z2, rowp, colp)
    h2, z3 = _tc2(acc2, z2, b2, W3, dis)
    acc3 = prop32(z3, rowp, colp)
    h3, z4 = _tc4(acc3, z3, b3, W4, dis)
    acc4 = _get_propagate(16)(z4, rowp, colp)
    xcpad, key = _tc5(acc4, z4, b4, dis, h1, h2, h3)

    pe, po = _get_sc_topk()(key, cnt, xcpad)

    c5 = jnp.concatenate([c5w[:, 0, :].T, jnp.zeros((31, 16), jnp.float32)], axis=0)
    w6 = jnp.transpose(c6w, (2, 1, 0)).reshape(80, 32)
    f1p = f1w.reshape(32, 11, 128).transpose(1, 0, 2).reshape(352, 128)

    return _tc_head(pe.reshape(G * 16, 128), po.reshape(G * 16, 128),
                    c5, c5b, w6, c6b, f1p, f1b, f2w, f2b)


# speedup vs baseline: 1.0024x; 1.0024x over previous
import functools

import jax
import jax.numpy as jnp
from jax import lax
from jax.experimental import pallas as pl
from jax.experimental.pallas import tpu as pltpu
from jax.experimental.pallas import tpu_sc as plsc

N = 10000
E = 320000
G = 100
K = 30
NPAD = 10240
GPAD = 128
NC = 2
NS = 16
NW = NC * NS
EROWS = 2560
EPAD = EROWS * 128
ROWS_PER_W = EROWS // NW
NROWS_PER_TILE = NPAD // NS


@functools.lru_cache(maxsize=None)
def _mesh():
    return plsc.VectorSubcoreMesh(core_axis_name="c", subcore_axis_name="s",
                                  num_cores=NC, num_subcores=NS)


def _zero_vmem_2d(ref, nrows, width):
    z16 = jnp.zeros((16,), jnp.float32)

    def body(i, _):
        for c0 in range(0, width, 16):
            ref[i, pl.ds(c0, 16)] = z16
        return 0

    lax.fori_loop(0, nrows, body, 0)


def _zero_vmem_1d_i32(ref, nelems):
    z16 = jnp.zeros((16,), jnp.int32)

    def body(i, _):
        ref[pl.ds(i * 16, 16)] = z16
        return 0

    lax.fori_loop(0, nelems // 16, body, 0)


_EREAL = E // 128


@functools.lru_cache(maxsize=None)
def _get_sc_pre():
    @functools.partial(
        pl.kernel,
        out_type=(
            jax.ShapeDtypeStruct((EROWS, 128), jnp.int32),
            jax.ShapeDtypeStruct((EROWS, 128), jnp.int32),
            jax.ShapeDtypeStruct((NW, NPAD), jnp.int32),
            jax.ShapeDtypeStruct((GPAD,), jnp.int32),
        ),
        mesh=_mesh(),
        compiler_params=pltpu.CompilerParams(needs_layout_passes=False, use_tc_tiling_on_sc=False),
        scratch_types=[
            pltpu.VMEM((ROWS_PER_W, 128), jnp.int32),
            pltpu.VMEM((ROWS_PER_W, 128), jnp.int32),
            pltpu.VMEM((NPAD,), jnp.int32),
            pltpu.VMEM((NPAD,), jnp.int32),
            pltpu.VMEM((GPAD,), jnp.int32),
        ],
    )
    def sc_pre(ei_hbm, batch_hbm, rowp_hbm, colp_hbm, degs_hbm, cnt_hbm,
               rowb, colb, degb, batchb, cntb):
        c = lax.axis_index("c")
        s = lax.axis_index("s")
        wid = s * NC + c
        base = wid * ROWS_PER_W
        nreal = _EREAL - 31 * ROWS_PER_W

        _zero_vmem_1d_i32(degb, NPAD)

        iota = lax.iota(jnp.int32, 16)
        dummy = N + iota
        ones = jnp.ones((16,), jnp.int32)

        @pl.when(wid < NW - 1)
        def _():
            pltpu.sync_copy(ei_hbm.at[0, pl.ds(base, ROWS_PER_W)], rowb)
            pltpu.sync_copy(ei_hbm.at[1, pl.ds(base, ROWS_PER_W)], colb)

        @pl.when(wid == NW - 1)
        def _():
            pltpu.sync_copy(ei_hbm.at[0, pl.ds(base, nreal)],
                            rowb.at[pl.ds(0, nreal)])
            pltpu.sync_copy(ei_hbm.at[1, pl.ds(base, nreal)],
                            colb.at[pl.ds(0, nreal)])

            def padrow(j, _):
                for k in range(8):
                    v = ((base + j) * 128 + k * 16 + iota) % N
                    rowb[j, pl.ds(k * 16, 16)] = v
                    colb[j, pl.ds(k * 16, 16)] = v
                return 0

            lax.fori_loop(nreal, ROWS_PER_W, padrow, 0)

        def erow(j, _):
            for k in range(8):
                rv = rowb[j, pl.ds(k * 16, 16)]
                cv = colb[j, pl.ds(k * 16, 16)]
                cpv = jnp.where(rv == cv, dummy, cv)
                colb[j, pl.ds(k * 16, 16)] = cpv
                plsc.addupdate_scatter(degb, [cpv], ones)
            return 0

        lax.fori_loop(0, ROWS_PER_W, erow, 0)

        pltpu.sync_copy(rowb, rowp_hbm.at[pl.ds(base, ROWS_PER_W)])
        pltpu.sync_copy(colb, colp_hbm.at[pl.ds(base, ROWS_PER_W)])
        pltpu.sync_copy(degb, degs_hbm.at[wid])

        @pl.when(jnp.logical_and(c == 0, s == 0))
        def _():
            _zero_vmem_1d_i32(cntb, GPAD)
            pltpu.sync_copy(batch_hbm, batchb.at[pl.ds(0, N)])

            def brow(i, _):
                bv = batchb[pl.ds(i * 16, 16)]
                plsc.addupdate_scatter(cntb, [bv], ones)
                return 0

            lax.fori_loop(0, N // 16, brow, 0)
            pltpu.sync_copy(cntb, cnt_hbm)

    return sc_pre


_GK = 10
_NGO = ROWS_PER_W // (2 * _GK)
_ZR = 80


@functools.lru_cache(maxsize=None)
def _get_propagate(F):
    @functools.partial(
        pl.kernel,
        out_type=jax.ShapeDtypeStruct((NC, NPAD, F), jnp.float32),
        mesh=_mesh(),
        compiler_params=pltpu.CompilerParams(needs_layout_passes=False, use_tc_tiling_on_sc=False),
        scratch_types=[
            pltpu.VMEM((ROWS_PER_W, 128), jnp.int32),
            pltpu.VMEM((ROWS_PER_W, 128), jnp.int32),
            pltpu.VMEM((2, _GK, 128, F), jnp.float32),
            pltpu.VMEM((_ZR, F), jnp.float32),
            pltpu.VMEM_SHARED((NPAD, F), jnp.float32),
            pltpu.SemaphoreType.DMA,
            pltpu.SemaphoreType.DMA,
        ],
    )
    def prop(z_hbm, row_hbm, colp_hbm, out_hbm, rowb, colb, gbuf, zbuf, acc,
             semg, sems):
        c = lax.axis_index("c")
        s = lax.axis_index("s")
        wid = s * NC + c
        base = wid * ROWS_PER_W
        stripe = pl.ds(s * NROWS_PER_TILE, NROWS_PER_TILE)

        _zero_vmem_2d(zbuf, _ZR, F)
        for q in range(NROWS_PER_TILE // _ZR):
            pltpu.sync_copy(
                zbuf, acc.at[pl.ds(s * NROWS_PER_TILE + q * _ZR, _ZR)])

        pltpu.sync_copy(row_hbm.at[pl.ds(base, ROWS_PER_W)], rowb)
        pltpu.sync_copy(colp_hbm.at[pl.ds(base, ROWS_PER_W)], colb)
        plsc.subcore_barrier()

        def start_gathers(grp, half):
            for k in range(_GK):
                pltpu.async_copy(z_hbm.at[rowb.at[grp * _GK + k]],
                                 gbuf.at[half, k], semg)

        def wait_gathers(grp, half):
            for k in range(_GK):
                pltpu.make_async_copy(z_hbm.at[rowb.at[grp * _GK + k]],
                                      gbuf.at[half, k], semg).wait()

        def fire_scatters(grp, half):
            for k in range(_GK):
                pltpu.async_copy(gbuf.at[half, k],
                                 acc.at[colb.at[grp * _GK + k]], sems,
                                 add=True)

        def drain_scatters(grp, half):
            for k in range(_GK):
                pltpu.make_async_copy(gbuf.at[half, k],
                                      acc.at[colb.at[grp * _GK + k]],
                                      sems).wait()

        start_gathers(0, 0)

        def outer(go, _):
            ga = 2 * go
            gb = 2 * go + 1
            wait_gathers(ga, 0)

            @pl.when(go > 0)
            def _():
                drain_scatters(ga - 1, 1)

            start_gathers(gb, 1)
            fire_scatters(ga, 0)
            wait_gathers(gb, 1)
            drain_scatters(ga, 0)

            @pl.when(go < _NGO - 1)
            def _():
                start_gathers(ga + 2, 0)

            fire_scatters(gb, 1)
            return 0

        lax.fori_loop(0, _NGO, outer, 0)
        drain_scatters(2 * _NGO - 1, 1)

        plsc.subcore_barrier()
        pltpu.sync_copy(acc.at[stripe], out_hbm.at[c, stripe])

    return prop


_NG_PER_W = (G + NW - 1) // NW


@functools.lru_cache(maxsize=None)
def _get_sc_topk():
    @functools.partial(
        pl.kernel,
        out_type=(
            jax.ShapeDtypeStruct((G, 16, 128), jnp.float32),
            jax.ShapeDtypeStruct((G, 16, 128), jnp.float32),
        ),
        mesh=_mesh(),
        compiler_params=pltpu.CompilerParams(needs_layout_passes=False, use_tc_tiling_on_sc=False),
        scratch_types=[
            pltpu.VMEM((NPAD,), jnp.float32),
            pltpu.VMEM((GPAD,), jnp.int32),
            pltpu.VMEM((GPAD,), jnp.int32),
            pltpu.VMEM((2, 16), jnp.int32),
            pltpu.VMEM((2, 16, 128), jnp.float32),
            pltpu.SemaphoreType.DMA,
        ],
    )
    def sc_topk(key_hbm, cnt_hbm, xc_hbm, pe_hbm, po_hbm,
                keyb, startb, cntb, idxb, gbuf, sem):
        c = lax.axis_index("c")
        s = lax.axis_index("s")
        wid = s * NC + c

        pltpu.sync_copy(key_hbm, keyb)
        pltpu.sync_copy(cnt_hbm, cntb)

        iota = lax.iota(jnp.int32, 16)

        def scan_chunk(i, carry):
            cv = cntb[pl.ds(i * 16, 16)]
            cs = plsc.cumsum(cv)
            startb[pl.ds(i * 16, 16)] = carry + cs - cv
            return carry + jnp.sum(jnp.where(iota == 15, cs, 0))

        lax.fori_loop(0, GPAD // 16, scan_chunk, jnp.int32(0))

        def do_graph(gl, _):
            g = wid + NW * gl

            @pl.when(g < G)
            def _():
                sel = iota == (g % 16)
                start = jnp.sum(jnp.where(sel, startb[pl.ds((g // 16) * 16, 16)], 0))
                cnt = jnp.sum(jnp.where(sel, cntb[pl.ds((g // 16) * 16, 16)], 0))
                nch = (cnt + 15) // 16

                idxb[0, pl.ds(0, 16)] = N + iota
                idxb[1, pl.ds(0, 16)] = N + iota
                def ichunk(i, _):
                    a = keyb[pl.ds(start + i * 16, 16)]
                    ii = i * 16 + iota
                    validi = ii < cnt
                    rank = jnp.zeros((16,), jnp.int32)

                    def jlt(jc, r):
                        b = keyb[pl.ds(start + jc * 16, 16)]
                        for l in range(16):
                            r = r + jnp.where(b[l] >= a, 1, 0)
                        return r

                    rank = lax.fori_loop(0, i, jlt, rank)

                    for l in range(16):
                        bl = a[l]
                        jj = i * 16 + l
                        take = ((bl > a) | ((bl == a) & (iota > l))) & (jj < cnt)
                        rank = rank + jnp.where(take, 1, 0)

                    def jgt(jc, r):
                        b = keyb[pl.ds(start + jc * 16, 16)]
                        for l in range(16):
                            r = r + jnp.where(b[l] > a, 1, 0)
                        return r

                    rank = lax.fori_loop(i + 1, nch - 1, jgt, rank)

                    def last(r):
                        b = keyb[pl.ds(start + (nch - 1) * 16, 16)]
                        for l in range(16):
                            jj = (nch - 1) * 16 + l
                            r = r + jnp.where((b[l] > a) & (jj < cnt), 1, 0)
                        return r

                    rank = lax.cond(i < nch - 1, last, lambda r: r, rank)
                    node = start + ii
                    half = rank // 2
                    even = (rank % 2) == 0
                    mask_sel = validi & (rank < K)
                    plsc.store_scatter(idxb, [jnp.zeros((16,), jnp.int32), half],
                                       node, mask=mask_sel & even)
                    plsc.store_scatter(idxb, [jnp.ones((16,), jnp.int32), half],
                                       node, mask=mask_sel & (~even))
                    return 0

                lax.fori_loop(0, nch, ichunk, 0)

                he = pltpu.async_copy(xc_hbm.at[idxb.at[0]], gbuf.at[0], sem)
                ho = pltpu.async_copy(xc_hbm.at[idxb.at[1]], gbuf.at[1], sem)
                he.wait()
                pltpu.sync_copy(gbuf.at[0], pe_hbm.at[g])
                ho.wait()
                pltpu.sync_copy(gbuf.at[1], po_hbm.at[g])

            return 0

        lax.fori_loop(0, _NG_PER_W, do_graph, 0)

    return sc_topk


def _tc1_body(x_ref, w1_ref, degs_ref, z1_ref, dis_ref):
    deg = jnp.sum(degs_ref[:, :N].astype(jnp.float32), axis=0) + 1.0
    dis = lax.rsqrt(deg)[:, None]
    dis_ref[...] = dis
    z1_ref[:N, :] = jnp.dot(x_ref[...], w1_ref[...],
                            preferred_element_type=jnp.float32) * dis
    z1_ref[N:, :] = jnp.zeros((NPAD - N, 32), jnp.float32)


_tc1 = pl.pallas_call(
    _tc1_body,
    out_shape=(
        jax.ShapeDtypeStruct((NPAD, 32), jnp.float32),
        jax.ShapeDtypeStruct((N, 1), jnp.float32),
    ),
)


def _tc_mid_body(accs_ref, z_ref, b_ref, w_ref, dis_ref, h_ref, znext_ref, *, fout):
    acc = accs_ref[0, :N, :] + accs_ref[1, :N, :] + z_ref[:N, :]
    h = jnp.tanh(dis_ref[...] * acc + b_ref[...][None, :])
    h_ref[...] = h
    z = jnp.dot(h, w_ref[...], preferred_element_type=jnp.float32) * dis_ref[...]
    if fout == 16:
        znext_ref[:N, :] = jnp.broadcast_to(z[:, :1], (N, 16))
        znext_ref[N:, :] = jnp.zeros((NPAD - N, 16), jnp.float32)
    else:
        znext_ref[:N, :] = z
        znext_ref[N:, :] = jnp.zeros((NPAD - N, 32), jnp.float32)


def _make_tc_mid(fin, fout):
    return pl.pallas_call(
        functools.partial(_tc_mid_body, fout=fout),
        out_shape=(
            jax.ShapeDtypeStruct((N, fin), jnp.float32),
            jax.ShapeDtypeStruct((NPAD, fout), jnp.float32),
        ),
    )


_tc2 = _make_tc_mid(32, 32)
_tc4 = _make_tc_mid(32, 16)


def _tc5_body(accs_ref, z4_ref, b4_ref, dis_ref, h1_ref, h2_ref, h3_ref,
              xc_ref, key_ref):
    acc = accs_ref[0, :N, :1] + accs_ref[1, :N, :1] + z4_ref[:N, :1]
    h4 = jnp.tanh(dis_ref[...] * acc + b4_ref[...][None, :])
    xc_ref[:N, 0:32] = h1_ref[...]
    xc_ref[:N, 32:64] = h2_ref[...]
    xc_ref[:N, 64:96] = h3_ref[...]
    xc_ref[:N, 96:128] = jnp.concatenate(
        [h4, jnp.zeros((N, 31), jnp.float32)], axis=1)
    xc_ref[N:NPAD, :] = jnp.zeros((NPAD - N, 128), jnp.float32)
    key_ref[pl.ds(0, N)] = h4[:, 0]
    key_ref[pl.ds(N, NPAD - N)] = jnp.zeros((NPAD - N,), jnp.float32)


_tc5 = pl.pallas_call(
    _tc5_body,
    out_shape=(
        jax.ShapeDtypeStruct((NPAD, 128), jnp.float32),
        jax.ShapeDtypeStruct((NPAD,), jnp.float32),
    ),
)


def _tc_head_body(pe_ref, po_ref, c5_ref, c5b_ref, w6_ref, c6b_ref,
                  f1_ref, f1b_ref, f2_ref, f2b_ref, out_ref):
    ae = jnp.maximum(jnp.dot(pe_ref[...], c5_ref[...],
                             preferred_element_type=jnp.float32)
                     + c5b_ref[...][None, :], 0.0)
    ao = jnp.maximum(jnp.dot(po_ref[...], c5_ref[...],
                             preferred_element_type=jnp.float32)
                     + c5b_ref[...][None, :], 0.0)
    b3 = jnp.maximum(ae, ao).reshape(G, 16, 16)
    h1 = jnp.zeros((G, 128), jnp.float32)
    for t in range(11):
        a = jnp.zeros((G, 32), jnp.float32)
        for dt in range(5):
            a = a + jnp.dot(b3[:, t + dt, :],
                            w6_ref[pl.ds(dt * 16, 16), :],
                            preferred_element_type=jnp.float32)
        c2t = jnp.maximum(a + c6b_ref[...][None, :], 0.0)
        h1 = h1 + jnp.dot(c2t, f1_ref[pl.ds(t * 32, 32), :],
                          preferred_element_type=jnp.float32)
    h = jnp.maximum(h1 + f1b_ref[...][None, :], 0.0)
    logits = jnp.dot(h, f2_ref[...],
                     preferred_element_type=jnp.float32) + f2b_ref[...][None, :]
    m = jnp.max(logits, axis=-1, keepdims=True)
    lse = m + jnp.log(jnp.sum(jnp.exp(logits - m), axis=-1, keepdims=True))
    out_ref[...] = logits - lse


_tc_head = pl.pallas_call(
    _tc_head_body,
    out_shape=jax.ShapeDtypeStruct((G, 2), jnp.float32),
)


def kernel(x, edge_index, batch, W1, b1, W2, b2, W3, b3, W4, b4,
           c5w, c5b, c6w, c6b, f1w, f1b, f2w, f2b):
    ei3 = edge_index.reshape(2, E // 128, 128)

    rowp, colp, degs, cnt = _get_sc_pre()(ei3, batch)
    z1, dis = _tc1(x, W1, degs)

    prop32 = _get_propagate(32)
    acc1 = prop32(z1, rowp, colp)
    h1, z2 = _tc2(acc1, z1, b1, W2, dis)
    acc2 = prop32(z2, rowp, colp)
    h2, z3 = _tc2(acc2, z2, b2, W3, dis)
    acc3 = prop32(z3, rowp, colp)
    h3, z4 = _tc4(acc3, z3, b3, W4, dis)
    acc4 = _get_propagate(16)(z4, rowp, colp)
    xcpad, key = _tc5(acc4, z4, b4, dis, h1, h2, h3)

    pe, po = _get_sc_topk()(key, cnt, xcpad)

    c5 = jnp.concatenate([c5w[:, 0, :].T, jnp.zeros((31, 16), jnp.float32)], axis=0)
    w6 = jnp.transpose(c6w, (2, 1, 0)).reshape(80, 32)
    f1p = f1w.reshape(32, 11, 128).transpose(1, 0, 2).reshape(352, 128)

    return _tc_head(pe.reshape(G * 16, 128), po.reshape(G * 16, 128),
                    c5, c5b, w6, c6b, f1p, f1b, f2w, f2b)
